# baseline (speedup 1.0000x reference)
import functools

import jax
import jax.numpy as jnp
from jax import lax
from jax.experimental import pallas as pl
from jax.experimental.pallas import tpu as pltpu
from jax.experimental.pallas import tpu_sc as plsc

_L = 16
_NC = 2
_NS = 16
_NW = _NC * _NS
_CHUNK = 4096


@functools.lru_cache(maxsize=None)
def _make_skew(bn):
    rows_w = bn // _NW
    nchunk = rows_w // _CHUNK
    mesh = plsc.VectorSubcoreMesh(core_axis_name="c", subcore_axis_name="s")

    @functools.partial(
        pl.kernel,
        out_type=jax.ShapeDtypeStruct((bn * 9,), jnp.float32),
        mesh=mesh,
        compiler_params=pltpu.CompilerParams(needs_layout_passes=False),
        scratch_types=[
            pltpu.VMEM((_CHUNK * 3,), jnp.float32),
            pltpu.VMEM((_CHUNK * 3,), jnp.float32),
            pltpu.VMEM((_CHUNK * 9,), jnp.float32),
            pltpu.VMEM((_CHUNK * 9,), jnp.float32),
            pltpu.SemaphoreType.DMA,
            pltpu.SemaphoreType.DMA,
            pltpu.SemaphoreType.DMA,
            pltpu.SemaphoreType.DMA,
        ],
    )
    def skew(dw_hbm, out_hbm, in_a, in_b, out_a, out_b, si_a, si_b, so_a, so_b):
        wid = lax.axis_index("s") * _NC + lax.axis_index("c")
        in_base = wid * (rows_w * 3)
        out_base = wid * (rows_w * 9)

        ins = (in_a, in_b)
        outs = (out_a, out_b)
        sis = (si_a, si_b)
        sos = (so_a, so_b)

        lane = lax.iota(jnp.int32, _L)
        gx, gy, gz = (lane * 3 + c for c in range(3))
        s_idx = [lane * 9 + j for j in range(9)]

        def compute(in_v, out_v, with_zero):
            zero = jnp.zeros((_L,), jnp.float32)

            def body(j, carry):
                ib = j * (3 * _L)
                ob = j * (9 * _L)
                x = plsc.load_gather(in_v, [gx + ib])
                y = plsc.load_gather(in_v, [gy + ib])
                z = plsc.load_gather(in_v, [gz + ib])
                plsc.store_scatter(out_v, [s_idx[1] + ob], -z)
                plsc.store_scatter(out_v, [s_idx[2] + ob], y)
                plsc.store_scatter(out_v, [s_idx[3] + ob], z)
                plsc.store_scatter(out_v, [s_idx[5] + ob], -x)
                plsc.store_scatter(out_v, [s_idx[6] + ob], -y)
                plsc.store_scatter(out_v, [s_idx[7] + ob], x)
                if with_zero:
                    plsc.store_scatter(out_v, [s_idx[0] + ob], zero)
                    plsc.store_scatter(out_v, [s_idx[4] + ob], zero)
                    plsc.store_scatter(out_v, [s_idx[8] + ob], zero)
                return carry

            lax.fori_loop(0, _CHUNK // _L, body, 0)

        def start_in(ci):
            b = ci % 2
            return pltpu.async_copy(
                dw_hbm.at[pl.ds(in_base + ci * (_CHUNK * 3), _CHUNK * 3)],
                ins[b], sis[b])

        def start_out(ci):
            b = ci % 2
            return pltpu.async_copy(
                outs[b],
                out_hbm.at[pl.ds(out_base + ci * (_CHUNK * 9), _CHUNK * 9)],
                sos[b])

        in_h = {}
        out_h = {}
        in_h[0] = start_in(0)
        for ci in range(nchunk):
            if ci + 1 < nchunk:
                in_h[ci + 1] = start_in(ci + 1)
            in_h[ci].wait()
            if ci >= 2:
                out_h[ci - 2].wait()
            compute(ins[ci % 2], outs[ci % 2], with_zero=(ci < min(2, nchunk)))
            out_h[ci] = start_out(ci)
        for ci in range(max(0, nchunk - 2), nchunk):
            out_h[ci].wait()

    return skew


def kernel(dw):
    bn = dw.shape[0]
    out = _make_skew(bn)(dw.reshape(bn * 3))
    return out.reshape(bn, 3, 3)


# speedup vs baseline: 24.0834x; 24.0834x over previous
import functools

import jax
import jax.numpy as jnp
from jax import lax
from jax.experimental import pallas as pl
from jax.experimental.pallas import tpu as pltpu
from jax.experimental.pallas import tpu_sc as plsc

_L = 16
_NW = 32
_CHUNK = 32

_PLANE = (
    ((1, 2, -1.0), (2, 1, 1.0)),
    ((0, 2, 1.0), (2, 0, -1.0)),
    ((0, 1, -1.0), (1, 0, 1.0)),
)


@functools.lru_cache(maxsize=None)
def _make_skew(cts):
    ct_w = cts // _NW
    nchunk = ct_w // _CHUNK
    mesh = plsc.VectorSubcoreMesh(core_axis_name="c", subcore_axis_name="s")

    @functools.partial(
        pl.kernel,
        out_type=jax.ShapeDtypeStruct((3, cts * 4, 128), jnp.float32),
        mesh=mesh,
        compiler_params=pltpu.CompilerParams(needs_layout_passes=False),
        scratch_types=[
            pltpu.VMEM((2, 3, _CHUNK, 128), jnp.float32),
            pltpu.VMEM((2, 3, _CHUNK * 4, 128), jnp.float32),
            pltpu.SemaphoreType.DMA,
            pltpu.SemaphoreType.DMA,
            pltpu.SemaphoreType.DMA,
            pltpu.SemaphoreType.DMA,
        ],
    )
    def skew(x_hbm, y_hbm, z_hbm, o_hbm, in_v, out_v, si_a, si_b, so_a, so_b):
        wid = lax.axis_index("s") * 2 + lax.axis_index("c")
        ct0 = wid * ct_w
        comps = (x_hbm, y_hbm, z_hbm)
        sis = (si_a, si_b)
        sos = (so_a, so_b)

        zero = jnp.zeros((_L,), jnp.float32)

        for s in range(2):
            for i in range(3):
                zrows = (i, 3) if i != 3 else (3,)

                def zbody(ct, carry, s=s, i=i, zrows=zrows):
                    for k in zrows:
                        for j in range(8):
                            out_v[s, i, ct * 4 + k, pl.ds(j * _L, _L)] = zero
                    return carry

                lax.fori_loop(0, _CHUNK, zbody, 0)

        def start_in(ci):
            s = ci % 2
            base = ct0 + ci * _CHUNK
            return [
                pltpu.async_copy(
                    comps[c].at[pl.ds(base, _CHUNK), :], in_v.at[s, c], sis[s])
                for c in range(3)
            ]

        def start_out(ci):
            s = ci % 2
            rbase = (ct0 + ci * _CHUNK) * 4
            return [
                pltpu.async_copy(
                    out_v.at[s, i], o_hbm.at[i, pl.ds(rbase, _CHUNK * 4), :],
                    sos[s])
                for i in range(3)
            ]

        def compute(s):
            for i in range(3):
                def body(ct, carry, s=s, i=i):
                    for k, c, sign in _PLANE[i]:
                        for j in range(8):
                            v = in_v[s, c, ct, pl.ds(j * _L, _L)]
                            if sign < 0:
                                v = -v
                            out_v[s, i, ct * 4 + k, pl.ds(j * _L, _L)] = v
                    return carry

                lax.fori_loop(0, _CHUNK, body, 0)

        in_h = {}
        out_h = {}
        in_h[0] = start_in(0)
        for ci in range(nchunk):
            if ci + 1 < nchunk:
                in_h[ci + 1] = start_in(ci + 1)
            for h in in_h.pop(ci):
                h.wait()
            if ci >= 2:
                for h in out_h.pop(ci - 2):
                    h.wait()
            compute(ci % 2)
            out_h[ci] = start_out(ci)
        for ci in sorted(out_h):
            for h in out_h[ci]:
                h.wait()

    return skew


def kernel(dw):
    bn = dw.shape[0]
    cts = bn // 128
    x = dw[:, 0].reshape(cts, 128)
    y = dw[:, 1].reshape(cts, 128)
    z = dw[:, 2].reshape(cts, 128)
    o = _make_skew(cts)(x, y, z)
    return (
        o.reshape(3, cts, 4, 128)
        .transpose(1, 3, 0, 2)[:, :, :, :3]
        .reshape(bn, 3, 3)
    )


# speedup vs baseline: 26.9222x; 1.1179x over previous
import functools

import jax
import jax.numpy as jnp
from jax import lax
from jax.experimental import pallas as pl
from jax.experimental.pallas import tpu as pltpu
from jax.experimental.pallas import tpu_sc as plsc

_L = 16
_NW = 32
_CHUNK = 32

_PLANE = (
    ((1, 2, -1.0), (2, 1, 1.0)),
    ((0, 2, 1.0), (2, 0, -1.0)),
    ((0, 1, -1.0), (1, 0, 1.0)),
)


@functools.lru_cache(maxsize=None)
def _make_skew(cts):
    ct_w = cts // _NW
    nchunk = ct_w // _CHUNK
    mesh = plsc.VectorSubcoreMesh(core_axis_name="c", subcore_axis_name="s")

    @functools.partial(
        pl.kernel,
        out_type=jax.ShapeDtypeStruct((3, cts * 3, 128), jnp.float32),
        mesh=mesh,
        compiler_params=pltpu.CompilerParams(needs_layout_passes=False),
        scratch_types=[
            pltpu.VMEM((2, 3, _CHUNK, 128), jnp.float32),
            pltpu.VMEM((2, 3, _CHUNK * 3, 128), jnp.float32),
            pltpu.SemaphoreType.DMA,
            pltpu.SemaphoreType.DMA,
            pltpu.SemaphoreType.DMA,
            pltpu.SemaphoreType.DMA,
        ],
    )
    def skew(x_hbm, y_hbm, z_hbm, o_hbm, in_v, out_v, si_a, si_b, so_a, so_b):
        wid = lax.axis_index("s") * 2 + lax.axis_index("c")
        ct0 = wid * ct_w
        comps = (x_hbm, y_hbm, z_hbm)
        sis = (si_a, si_b)
        sos = (so_a, so_b)

        zero = jnp.zeros((_L,), jnp.float32)

        for s in range(2):
            for i in range(3):

                @functools.partial(plsc.parallel_loop, 0, _CHUNK, unroll=2)
                def zbody(ct, s=s, i=i):
                    for j in range(8):
                        out_v[s, i, ct * 3 + i, pl.ds(j * _L, _L)] = zero

        def start_in(ci):
            s = ci % 2
            base = ct0 + ci * _CHUNK
            return [
                pltpu.async_copy(
                    comps[c].at[pl.ds(base, _CHUNK), :], in_v.at[s, c], sis[s])
                for c in range(3)
            ]

        def start_out(ci):
            s = ci % 2
            rbase = (ct0 + ci * _CHUNK) * 3
            return [
                pltpu.async_copy(
                    out_v.at[s, i], o_hbm.at[i, pl.ds(rbase, _CHUNK * 3), :],
                    sos[s])
                for i in range(3)
            ]

        def compute(s):
            for i in range(3):

                @functools.partial(plsc.parallel_loop, 0, _CHUNK, unroll=2)
                def body(ct, s=s, i=i):
                    vals = []
                    for k, c, sign in _PLANE[i]:
                        for j in range(8):
                            v = in_v[s, c, ct, pl.ds(j * _L, _L)]
                            vals.append((k, j, -v if sign < 0 else v))
                    for k, j, v in vals:
                        out_v[s, i, ct * 3 + k, pl.ds(j * _L, _L)] = v

        in_h = {}
        out_h = {}
        in_h[0] = start_in(0)
        for ci in range(nchunk):
            if ci + 1 < nchunk:
                in_h[ci + 1] = start_in(ci + 1)
            for h in in_h.pop(ci):
                h.wait()
            if ci >= 2:
                for h in out_h.pop(ci - 2):
                    h.wait()
            compute(ci % 2)
            out_h[ci] = start_out(ci)
        for ci in sorted(out_h):
            for h in out_h[ci]:
                h.wait()

    return skew


def kernel(dw):
    bn = dw.shape[0]
    cts = bn // 128
    x = dw[:, 0].reshape(cts, 128)
    y = dw[:, 1].reshape(cts, 128)
    z = dw[:, 2].reshape(cts, 128)
    o = _make_skew(cts)(x, y, z)
    return (
        o.reshape(3, cts, 3, 128)
        .transpose(1, 3, 0, 2)
        .reshape(bn, 3, 3)
    )


# speedup vs baseline: 33.6031x; 1.2482x over previous
import functools

import jax
import jax.numpy as jnp
from jax import lax
from jax.experimental import pallas as pl
from jax.experimental.pallas import tpu as pltpu
from jax.experimental.pallas import tpu_sc as plsc

_L = 16
_NW = 32
_CHUNK = 32

_PLANE = (
    ((1, 2, -1.0), (2, 1, 1.0)),
    ((0, 2, 1.0), (2, 0, -1.0)),
    ((0, 1, -1.0), (1, 0, 1.0)),
)


@functools.lru_cache(maxsize=None)
def _make_skew(cts):
    ct_w = cts // _NW
    nchunk = ct_w // _CHUNK
    mesh = plsc.VectorSubcoreMesh(core_axis_name="c", subcore_axis_name="s")

    @functools.partial(
        pl.kernel,
        out_type=jax.ShapeDtypeStruct((3, cts * 4, 128), jnp.float32),
        mesh=mesh,
        compiler_params=pltpu.CompilerParams(needs_layout_passes=False),
        scratch_types=[
            pltpu.VMEM((2, 3, _CHUNK, 128), jnp.float32),
            pltpu.VMEM((2, 3, _CHUNK * 4, 128), jnp.float32),
            pltpu.SemaphoreType.DMA,
            pltpu.SemaphoreType.DMA,
            pltpu.SemaphoreType.DMA,
            pltpu.SemaphoreType.DMA,
        ],
    )
    def skew(x_hbm, y_hbm, z_hbm, o_hbm, in_v, out_v, si_a, si_b, so_a, so_b):
        wid = lax.axis_index("s") * 2 + lax.axis_index("c")
        ct0 = wid * ct_w
        comps = (x_hbm, y_hbm, z_hbm)
        sis = (si_a, si_b)
        sos = (so_a, so_b)

        zero = jnp.zeros((_L,), jnp.float32)

        for s in range(2):
            for i in range(3):

                @functools.partial(plsc.parallel_loop, 0, _CHUNK, unroll=2)
                def zbody(ct, s=s, i=i):
                    for k in (i, 3):
                        for j in range(8):
                            out_v[s, i, ct * 4 + k, pl.ds(j * _L, _L)] = zero

        def start_in(ci):
            s = ci % 2
            base = ct0 + ci * _CHUNK
            return [
                pltpu.async_copy(
                    comps[c].at[pl.ds(base, _CHUNK), :], in_v.at[s, c], sis[s])
                for c in range(3)
            ]

        def start_out(ci):
            s = ci % 2
            rbase = (ct0 + ci * _CHUNK) * 4
            return [
                pltpu.async_copy(
                    out_v.at[s, i], o_hbm.at[i, pl.ds(rbase, _CHUNK * 4), :],
                    sos[s])
                for i in range(3)
            ]

        def compute(s):
            for i in range(3):

                @functools.partial(plsc.parallel_loop, 0, _CHUNK, unroll=2)
                def body(ct, s=s, i=i):
                    vals = []
                    for k, c, sign in _PLANE[i]:
                        for j in range(8):
                            v = in_v[s, c, ct, pl.ds(j * _L, _L)]
                            vals.append((k, j, -v if sign < 0 else v))
                    for k, j, v in vals:
                        out_v[s, i, ct * 4 + k, pl.ds(j * _L, _L)] = v

        in_h = {}
        out_h = {}
        in_h[0] = start_in(0)
        for ci in range(nchunk):
            if ci + 1 < nchunk:
                in_h[ci + 1] = start_in(ci + 1)
            for h in in_h.pop(ci):
                h.wait()
            if ci >= 2:
                for h in out_h.pop(ci - 2):
                    h.wait()
            compute(ci % 2)
            out_h[ci] = start_out(ci)
        for ci in sorted(out_h):
            for h in out_h[ci]:
                h.wait()

    return skew


def kernel(dw):
    bn = dw.shape[0]
    cts = bn // 128
    x = dw[:, 0].reshape(cts, 128)
    y = dw[:, 1].reshape(cts, 128)
    z = dw[:, 2].reshape(cts, 128)
    o = _make_skew(cts)(x, y, z)
    return (
        o.reshape(3, cts, 4, 128)
        .transpose(1, 3, 0, 2)[:, :, :, :3]
        .reshape(bn, 3, 3)
    )
